# Initial kernel scaffold; baseline (speedup 1.0000x reference)
import jax
import jax.numpy as jnp
from jax.experimental import pallas as pl


def kernel(X, edge_index, conv_W0, conv_b0, conv_W1, conv_b1, conv_W2, conv_b2, conv_W3, conv_b3, conv_W4, conv_b4, conv_W5, conv_b5, conv_W6, conv_b6, conv_W7, conv_b7, cls_W0, cls_b0, cls_W1, cls_b1, cls_W2, cls_b2):
    raise NotImplementedError("write your pallas kernel here")


# speedup vs baseline: 6.1957x; 6.1957x over previous
import functools

import jax
import jax.numpy as jnp
from jax import lax
from jax.experimental import pallas as pl
from jax.experimental.pallas import tpu as pltpu
from jax.experimental.pallas import tpu_sc as plsc

f32 = jnp.float32
N = 10000
E = 160000
DIN = 128
DH = 512
NCLS = 64

NTILES = 16
BLK = 125
EPT = E // NTILES
NBLK = EPT // BLK
NPAD = 10240
ROWS_PT = NPAD // NTILES
CH = 8
CW = DH // CH
CPS = CH // 2
ZBLK = 128
ZCHUNKS = ROWS_PT // ZBLK
DEG_BLKS = NBLK // 2

_mesh = plsc.VectorSubcoreMesh(core_axis_name="c", subcore_axis_name="s")


@functools.partial(
    pl.kernel,
    out_type=(jax.ShapeDtypeStruct((NPAD, 16), f32),
              jax.ShapeDtypeStruct((NPAD, 16), f32)),
    mesh=_mesh,
    scratch_types=[
        pltpu.VMEM((NBLK, BLK), jnp.int32),
        pltpu.VMEM((BLK, 16), f32),
        pltpu.VMEM_SHARED((NPAD, 16), f32),
    ],
)
def _deg_kernel(dst_hbm, ones_hbm, zrow_hbm, deg0_hbm, deg1_hbm,
                dst_v, ones_v, dacc_sh):
    cid = lax.axis_index("c")
    sid = lax.axis_index("s")
    base = sid * ROWS_PT
    pltpu.sync_copy(zrow_hbm, dacc_sh.at[pl.ds(base, ROWS_PT)])
    pltpu.sync_copy(dst_hbm.at[sid], dst_v)
    pltpu.sync_copy(ones_hbm, ones_v)
    plsc.subcore_barrier()

    def body(j, c):
        pltpu.sync_copy(ones_v, dacc_sh.at[dst_v.at[cid * DEG_BLKS + j]],
                        add=True)
        return c

    lax.fori_loop(0, DEG_BLKS, body, 0)
    plsc.subcore_barrier()

    @pl.when(cid == 0)
    def _():
        pltpu.sync_copy(dacc_sh.at[pl.ds(base, ROWS_PT)],
                        deg0_hbm.at[pl.ds(base, ROWS_PT)])

    @pl.when(cid == 1)
    def _():
        pltpu.sync_copy(dacc_sh.at[pl.ds(base, ROWS_PT)],
                        deg1_hbm.at[pl.ds(base, ROWS_PT)])


@functools.partial(
    pl.kernel,
    out_type=tuple(jax.ShapeDtypeStruct((NPAD, CW), f32) for _ in range(CH)),
    mesh=_mesh,
    scratch_types=[
        pltpu.VMEM((NBLK, BLK), jnp.int32),
        pltpu.VMEM((NBLK, BLK), jnp.int32),
        pltpu.VMEM((BLK, CW), f32),
        pltpu.VMEM((ZBLK, CW), f32),
        pltpu.VMEM_SHARED((NPAD, CW), f32),
        pltpu.SemaphoreType.DMA,
    ],
    compiler_params=pltpu.CompilerParams(use_tc_tiling_on_sc=False),
)
def _agg_kernel(hp0, hp1, hp2, hp3, hp4, hp5, hp6, hp7,
                zrow_hbm, src_hbm, dst_hbm,
                a0, a1, a2, a3, a4, a5, a6, a7,
                src_v, dst_v, rows_v, zero_v, acc_sh, sem):
    cid = lax.axis_index("c")
    sid = lax.axis_index("s")
    base = sid * ROWS_PT
    pltpu.sync_copy(src_hbm.at[sid], src_v)
    pltpu.sync_copy(dst_hbm.at[sid], dst_v)
    pltpu.sync_copy(zrow_hbm, zero_v)

    def run_chunk(hp_hbm):
        def body(blk, c):
            pltpu.async_copy(hp_hbm.at[src_v.at[blk]], rows_v, sem).wait()
            pltpu.sync_copy(rows_v, acc_sh.at[dst_v.at[blk]], add=True)
            return c

        lax.fori_loop(0, NBLK, body, 0)

    hps = ((hp0, hp1, hp2, hp3), (hp4, hp5, hp6, hp7))
    outs = ((a0, a1, a2, a3), (a4, a5, a6, a7))
    for kc in range(CPS):
        for r in range(ZCHUNKS):
            pltpu.sync_copy(zero_v, acc_sh.at[pl.ds(base + r * ZBLK, ZBLK)])
        plsc.subcore_barrier()

        @pl.when(cid == 0)
        def _(kc=kc):
            run_chunk(hps[0][kc])

        @pl.when(cid == 1)
        def _(kc=kc):
            run_chunk(hps[1][kc])

        plsc.subcore_barrier()

        @pl.when(cid == 0)
        def _(kc=kc):
            pltpu.sync_copy(acc_sh.at[pl.ds(base, ROWS_PT)],
                            outs[0][kc].at[pl.ds(base, ROWS_PT)])

        @pl.when(cid == 1)
        def _(kc=kc):
            pltpu.sync_copy(acc_sh.at[pl.ds(base, ROWS_PT)],
                            outs[1][kc].at[pl.ds(base, ROWS_PT)])

        plsc.subcore_barrier()


NB = 1000
GRID = N // NB


def _k0_body(deg0, deg1, x, w0, dinv_out, *houts):
    deg = 1.0 + deg0[:, 0:1] + deg1[:, 0:1]
    dinv = lax.rsqrt(deg)
    dinv_out[...] = jnp.broadcast_to(dinv, (NB, 128))
    z = jnp.dot(x[...], w0[...], preferred_element_type=f32)
    hp = dinv * z
    for c, ref in enumerate(houts):
        ref[...] = hp[:, c * CW:(c + 1) * CW]


_k0 = pl.pallas_call(
    _k0_body,
    grid=(GRID,),
    in_specs=[
        pl.BlockSpec((NB, 16), lambda n: (n, 0)),
        pl.BlockSpec((NB, 16), lambda n: (n, 0)),
        pl.BlockSpec((NB, DIN), lambda n: (n, 0)),
        pl.BlockSpec((DIN, DH), lambda n: (0, 0)),
    ],
    out_specs=[pl.BlockSpec((NB, 128), lambda n: (n, 0))]
    + [pl.BlockSpec((NB, CW), lambda n: (n, 0))] * CH,
    out_shape=[jax.ShapeDtypeStruct((N, 128), f32)]
    + [jax.ShapeDtypeStruct((N, CW), f32)] * CH,
)


def _relu_rows(accs, hps, dv, b):
    parts = []
    for c in range(CH):
        u = (dv[:, :CW] * (accs[c][...] + hps[c][...])
             + b[:, c * CW:(c + 1) * CW])
        parts.append(jnp.maximum(u, 0.0))
    return jnp.concatenate(parts, axis=1)


def _mid_body(*refs):
    accs, hps = refs[:CH], refs[CH:2 * CH]
    dinv, b, w = refs[2 * CH], refs[2 * CH + 1], refs[2 * CH + 2]
    outs = refs[2 * CH + 3:]
    dv = dinv[...]
    u = _relu_rows(accs, hps, dv, b[...])
    z = jnp.dot(u, w[...], preferred_element_type=f32)
    for c, o in enumerate(outs):
        o[...] = dv[:, :CW] * z[:, c * CW:(c + 1) * CW]


_mid = pl.pallas_call(
    _mid_body,
    grid=(GRID,),
    in_specs=[pl.BlockSpec((NB, CW), lambda n: (n, 0))] * (2 * CH)
    + [pl.BlockSpec((NB, 128), lambda n: (n, 0)),
       pl.BlockSpec((1, DH), lambda n: (0, 0)),
       pl.BlockSpec((DH, DH), lambda n: (0, 0))],
    out_specs=[pl.BlockSpec((NB, CW), lambda n: (n, 0))] * CH,
    out_shape=[jax.ShapeDtypeStruct((N, CW), f32)] * CH,
)


def _fin_body(*refs):
    accs, hps = refs[:CH], refs[CH:2 * CH]
    dinv, b7 = refs[2 * CH], refs[2 * CH + 1]
    w0, b0, w1, b1, w2, b2 = refs[2 * CH + 2:2 * CH + 8]
    out = refs[2 * CH + 8]
    dv = dinv[...]
    u = _relu_rows(accs, hps, dv, b7[...])
    t = jnp.maximum(jnp.dot(u, w0[...], preferred_element_type=f32) + b0[...],
                    0.0)
    t = jnp.maximum(jnp.dot(t, w1[...], preferred_element_type=f32) + b1[...],
                    0.0)
    out[...] = jnp.dot(t, w2[...], preferred_element_type=f32) + b2[...]


_fin = pl.pallas_call(
    _fin_body,
    grid=(GRID,),
    in_specs=[pl.BlockSpec((NB, CW), lambda n: (n, 0))] * (2 * CH)
    + [pl.BlockSpec((NB, 128), lambda n: (n, 0)),
       pl.BlockSpec((1, DH), lambda n: (0, 0)),
       pl.BlockSpec((DH, DH), lambda n: (0, 0)),
       pl.BlockSpec((1, DH), lambda n: (0, 0)),
       pl.BlockSpec((DH, DH), lambda n: (0, 0)),
       pl.BlockSpec((1, DH), lambda n: (0, 0)),
       pl.BlockSpec((DH, NCLS), lambda n: (0, 0)),
       pl.BlockSpec((1, NCLS), lambda n: (0, 0))],
    out_specs=pl.BlockSpec((NB, NCLS), lambda n: (n, 0)),
    out_shape=jax.ShapeDtypeStruct((N, NCLS), f32),
)


def kernel(X, edge_index,
           conv_W0, conv_b0, conv_W1, conv_b1, conv_W2, conv_b2,
           conv_W3, conv_b3, conv_W4, conv_b4, conv_W5, conv_b5,
           conv_W6, conv_b6, conv_W7, conv_b7,
           cls_W0, cls_b0, cls_W1, cls_b1, cls_W2, cls_b2):
    Ws = [conv_W0, conv_W1, conv_W2, conv_W3,
          conv_W4, conv_W5, conv_W6, conv_W7]
    bs = [conv_b0, conv_b1, conv_b2, conv_b3,
          conv_b4, conv_b5, conv_b6, conv_b7]

    src_r = edge_index[0].reshape(NTILES, NBLK, BLK)
    dst_r = edge_index[1].reshape(NTILES, NBLK, BLK)
    ones_deg = jnp.ones((BLK, 16), f32)
    zdeg = jnp.zeros((ROWS_PT, 16), f32)
    zrow = jnp.zeros((ZBLK, CW), f32)

    deg0, deg1 = _deg_kernel(dst_r, ones_deg, zdeg)
    outs0 = _k0(deg0, deg1, X, Ws[0])
    dinv_b, hp = outs0[0], tuple(outs0[1:])
    for i in range(1, 8):
        acc = _agg_kernel(*hp, zrow, src_r, dst_r)
        hp = tuple(_mid(*acc, *hp, dinv_b, bs[i - 1].reshape(1, DH), Ws[i]))
    acc = _agg_kernel(*hp, zrow, src_r, dst_r)
    return _fin(*acc, *hp, dinv_b, bs[7].reshape(1, DH),
                cls_W0, cls_b0.reshape(1, DH),
                cls_W1, cls_b1.reshape(1, DH),
                cls_W2, cls_b2.reshape(1, NCLS))


# speedup vs baseline: 7.6791x; 1.2394x over previous
import functools

import jax
import jax.numpy as jnp
from jax import lax
from jax.experimental import pallas as pl
from jax.experimental.pallas import tpu as pltpu
from jax.experimental.pallas import tpu_sc as plsc

f32 = jnp.float32
N = 10000
E = 160000
DIN = 128
DH = 512
NCLS = 64

NTILES = 16
BLK = 125
EPT = E // NTILES
NBLK = EPT // BLK
NPAD = 10240
ROWS_PT = NPAD // NTILES
CH = 8
CW = DH // CH
CPS = CH // 2
ZBLK = 128
ZCHUNKS = ROWS_PT // ZBLK
DEG_BLKS = NBLK // 2

_mesh = plsc.VectorSubcoreMesh(core_axis_name="c", subcore_axis_name="s")


@functools.partial(
    pl.kernel,
    out_type=(jax.ShapeDtypeStruct((NPAD, 16), f32),
              jax.ShapeDtypeStruct((NPAD, 16), f32)),
    mesh=_mesh,
    scratch_types=[
        pltpu.VMEM((DEG_BLKS, BLK), jnp.int32),
        pltpu.VMEM((BLK, 16), f32),
        pltpu.VMEM_SHARED((NPAD, 16), f32),
    ],
    compiler_params=pltpu.CompilerParams(use_tc_tiling_on_sc=False),
)
def _deg_kernel(dst_hbm, ones_hbm, zrow_hbm, deg0_hbm, deg1_hbm,
                dst_v, ones_v, dacc_sh):
    cid = lax.axis_index("c")
    sid = lax.axis_index("s")
    base = sid * ROWS_PT
    pltpu.sync_copy(zrow_hbm, dacc_sh.at[pl.ds(base, ROWS_PT)])
    pltpu.sync_copy(dst_hbm.at[sid].at[pl.ds(cid * DEG_BLKS, DEG_BLKS)],
                    dst_v)
    pltpu.sync_copy(ones_hbm, ones_v)
    plsc.subcore_barrier()

    def body(j, c):
        pltpu.sync_copy(ones_v, dacc_sh.at[dst_v.at[j]], add=True)
        return c

    lax.fori_loop(0, DEG_BLKS, body, 0)
    plsc.subcore_barrier()

    @pl.when(cid == 0)
    def _():
        pltpu.sync_copy(dacc_sh.at[pl.ds(base, ROWS_PT)],
                        deg0_hbm.at[pl.ds(base, ROWS_PT)])

    @pl.when(cid == 1)
    def _():
        pltpu.sync_copy(dacc_sh.at[pl.ds(base, ROWS_PT)],
                        deg1_hbm.at[pl.ds(base, ROWS_PT)])


@functools.partial(
    pl.kernel,
    out_type=tuple(jax.ShapeDtypeStruct((NPAD, CW), f32) for _ in range(CH)),
    mesh=_mesh,
    scratch_types=[
        pltpu.VMEM((NBLK, BLK), jnp.int32),
        pltpu.VMEM((NBLK, BLK), jnp.int32),
        pltpu.VMEM((BLK, CW), f32),
        pltpu.VMEM((BLK, CW), f32),
        pltpu.VMEM((ZBLK, CW), f32),
        pltpu.VMEM_SHARED((NPAD, CW), f32),
        pltpu.SemaphoreType.DMA,
    ],
    compiler_params=pltpu.CompilerParams(use_tc_tiling_on_sc=False),
)
def _agg_kernel(hp0, hp1, hp2, hp3, hp4, hp5, hp6, hp7,
                zrow_hbm, src_hbm, dst_hbm,
                a0, a1, a2, a3, a4, a5, a6, a7,
                src_v, dst_v, rows0_v, rows1_v, zero_v, acc_sh, sem):
    cid = lax.axis_index("c")
    sid = lax.axis_index("s")
    base = sid * ROWS_PT
    pltpu.sync_copy(src_hbm.at[sid], src_v)
    pltpu.sync_copy(dst_hbm.at[sid], dst_v)
    pltpu.sync_copy(zrow_hbm, zero_v)

    def run_chunk(hp_hbm):
        pltpu.async_copy(hp_hbm.at[src_v.at[0]], rows0_v, sem)

        def body(p, c):
            b0 = 2 * p
            b1 = 2 * p + 1
            b2 = jnp.minimum(2 * p + 2, NBLK - 1)
            pltpu.make_async_copy(hp_hbm.at[src_v.at[b0]], rows0_v,
                                  sem).wait()
            pltpu.async_copy(hp_hbm.at[src_v.at[b1]], rows1_v, sem)
            pltpu.sync_copy(rows0_v, acc_sh.at[dst_v.at[b0]], add=True)
            pltpu.make_async_copy(hp_hbm.at[src_v.at[b1]], rows1_v,
                                  sem).wait()
            pltpu.async_copy(hp_hbm.at[src_v.at[b2]], rows0_v, sem)
            pltpu.sync_copy(rows1_v, acc_sh.at[dst_v.at[b1]], add=True)
            return c

        lax.fori_loop(0, NBLK // 2, body, 0)
        pltpu.make_async_copy(hp_hbm.at[src_v.at[NBLK - 1]], rows0_v,
                              sem).wait()

    hps = ((hp0, hp1, hp2, hp3), (hp4, hp5, hp6, hp7))
    outs = ((a0, a1, a2, a3), (a4, a5, a6, a7))
    for kc in range(CPS):
        for r in range(ZCHUNKS):
            pltpu.sync_copy(zero_v, acc_sh.at[pl.ds(base + r * ZBLK, ZBLK)])
        plsc.subcore_barrier()

        @pl.when(cid == 0)
        def _(kc=kc):
            run_chunk(hps[0][kc])

        @pl.when(cid == 1)
        def _(kc=kc):
            run_chunk(hps[1][kc])

        plsc.subcore_barrier()

        @pl.when(cid == 0)
        def _(kc=kc):
            pltpu.sync_copy(acc_sh.at[pl.ds(base, ROWS_PT)],
                            outs[0][kc].at[pl.ds(base, ROWS_PT)])

        @pl.when(cid == 1)
        def _(kc=kc):
            pltpu.sync_copy(acc_sh.at[pl.ds(base, ROWS_PT)],
                            outs[1][kc].at[pl.ds(base, ROWS_PT)])

        plsc.subcore_barrier()


NB = 1000
GRID = N // NB


def _k0_body(deg0, deg1, x, w0, dinv_out, *houts):
    deg = 1.0 + deg0[:, 0:1] + deg1[:, 0:1]
    dinv = lax.rsqrt(deg)
    dinv_out[...] = jnp.broadcast_to(dinv, (NB, 128))
    z = jnp.dot(x[...], w0[...], preferred_element_type=f32)
    hp = dinv * z
    for c, ref in enumerate(houts):
        ref[...] = hp[:, c * CW:(c + 1) * CW]


_k0 = pl.pallas_call(
    _k0_body,
    grid=(GRID,),
    in_specs=[
        pl.BlockSpec((NB, 16), lambda n: (n, 0)),
        pl.BlockSpec((NB, 16), lambda n: (n, 0)),
        pl.BlockSpec((NB, DIN), lambda n: (n, 0)),
        pl.BlockSpec((DIN, DH), lambda n: (0, 0)),
    ],
    out_specs=[pl.BlockSpec((NB, 128), lambda n: (n, 0))]
    + [pl.BlockSpec((NB, CW), lambda n: (n, 0))] * CH,
    out_shape=[jax.ShapeDtypeStruct((N, 128), f32)]
    + [jax.ShapeDtypeStruct((N, CW), f32)] * CH,
)


def _relu_rows(accs, hps, dv, b):
    parts = []
    for c in range(CH):
        u = (dv[:, :CW] * (accs[c][...] + hps[c][...])
             + b[:, c * CW:(c + 1) * CW])
        parts.append(jnp.maximum(u, 0.0))
    return jnp.concatenate(parts, axis=1)


def _mid_body(*refs):
    accs, hps = refs[:CH], refs[CH:2 * CH]
    dinv, b, w = refs[2 * CH], refs[2 * CH + 1], refs[2 * CH + 2]
    outs = refs[2 * CH + 3:]
    dv = dinv[...]
    u = _relu_rows(accs, hps, dv, b[...])
    z = jnp.dot(u, w[...], preferred_element_type=f32)
    for c, o in enumerate(outs):
        o[...] = dv[:, :CW] * z[:, c * CW:(c + 1) * CW]


_mid = pl.pallas_call(
    _mid_body,
    grid=(GRID,),
    in_specs=[pl.BlockSpec((NB, CW), lambda n: (n, 0))] * (2 * CH)
    + [pl.BlockSpec((NB, 128), lambda n: (n, 0)),
       pl.BlockSpec((1, DH), lambda n: (0, 0)),
       pl.BlockSpec((DH, DH), lambda n: (0, 0))],
    out_specs=[pl.BlockSpec((NB, CW), lambda n: (n, 0))] * CH,
    out_shape=[jax.ShapeDtypeStruct((N, CW), f32)] * CH,
)


def _fin_body(*refs):
    accs, hps = refs[:CH], refs[CH:2 * CH]
    dinv, b7 = refs[2 * CH], refs[2 * CH + 1]
    w0, b0, w1, b1, w2, b2 = refs[2 * CH + 2:2 * CH + 8]
    out = refs[2 * CH + 8]
    dv = dinv[...]
    u = _relu_rows(accs, hps, dv, b7[...])
    t = jnp.maximum(jnp.dot(u, w0[...], preferred_element_type=f32) + b0[...],
                    0.0)
    t = jnp.maximum(jnp.dot(t, w1[...], preferred_element_type=f32) + b1[...],
                    0.0)
    out[...] = jnp.dot(t, w2[...], preferred_element_type=f32) + b2[...]


_fin = pl.pallas_call(
    _fin_body,
    grid=(GRID,),
    in_specs=[pl.BlockSpec((NB, CW), lambda n: (n, 0))] * (2 * CH)
    + [pl.BlockSpec((NB, 128), lambda n: (n, 0)),
       pl.BlockSpec((1, DH), lambda n: (0, 0)),
       pl.BlockSpec((DH, DH), lambda n: (0, 0)),
       pl.BlockSpec((1, DH), lambda n: (0, 0)),
       pl.BlockSpec((DH, DH), lambda n: (0, 0)),
       pl.BlockSpec((1, DH), lambda n: (0, 0)),
       pl.BlockSpec((DH, NCLS), lambda n: (0, 0)),
       pl.BlockSpec((1, NCLS), lambda n: (0, 0))],
    out_specs=pl.BlockSpec((NB, NCLS), lambda n: (n, 0)),
    out_shape=jax.ShapeDtypeStruct((N, NCLS), f32),
)


def kernel(X, edge_index,
           conv_W0, conv_b0, conv_W1, conv_b1, conv_W2, conv_b2,
           conv_W3, conv_b3, conv_W4, conv_b4, conv_W5, conv_b5,
           conv_W6, conv_b6, conv_W7, conv_b7,
           cls_W0, cls_b0, cls_W1, cls_b1, cls_W2, cls_b2):
    Ws = [conv_W0, conv_W1, conv_W2, conv_W3,
          conv_W4, conv_W5, conv_W6, conv_W7]
    bs = [conv_b0, conv_b1, conv_b2, conv_b3,
          conv_b4, conv_b5, conv_b6, conv_b7]

    src_r = edge_index[0].reshape(NTILES, NBLK, BLK)
    dst_r = edge_index[1].reshape(NTILES, NBLK, BLK)
    ones_deg = jnp.ones((BLK, 16), f32)
    zdeg = jnp.zeros((ROWS_PT, 16), f32)
    zrow = jnp.zeros((ZBLK, CW), f32)

    deg0, deg1 = _deg_kernel(dst_r, ones_deg, zdeg)
    outs0 = _k0(deg0, deg1, X, Ws[0])
    dinv_b, hp = outs0[0], tuple(outs0[1:])
    for i in range(1, 8):
        acc = _agg_kernel(*hp, zrow, src_r, dst_r)
        hp = tuple(_mid(*acc, *hp, dinv_b, bs[i - 1].reshape(1, DH), Ws[i]))
    acc = _agg_kernel(*hp, zrow, src_r, dst_r)
    return _fin(*acc, *hp, dinv_b, bs[7].reshape(1, DH),
                cls_W0, cls_b0.reshape(1, DH),
                cls_W1, cls_b1.reshape(1, DH),
                cls_W2, cls_b2.reshape(1, NCLS))


# speedup vs baseline: 11.6978x; 1.5233x over previous
import functools

import jax
import jax.numpy as jnp
from jax import lax
from jax.experimental import pallas as pl
from jax.experimental.pallas import tpu as pltpu
from jax.experimental.pallas import tpu_sc as plsc

f32 = jnp.float32
N = 10000
E = 160000
DIN = 128
DH = 512
NCLS = 64

NTILES = 16
BLK = 125
EPT = E // NTILES
NBLK = EPT // BLK
NPAD = 10240
ROWS_PT = NPAD // NTILES
CH = 8
CW = DH // CH
CPS = CH // 2
ZBLK = 128
ZCHUNKS = ROWS_PT // ZBLK
DEG_BLKS = NBLK // 2

_mesh = plsc.VectorSubcoreMesh(core_axis_name="c", subcore_axis_name="s")


@functools.partial(
    pl.kernel,
    out_type=(jax.ShapeDtypeStruct((NPAD, 16), f32),
              jax.ShapeDtypeStruct((NPAD, 16), f32)),
    mesh=_mesh,
    scratch_types=[
        pltpu.VMEM((DEG_BLKS, BLK), jnp.int32),
        pltpu.VMEM((BLK, 16), f32),
        pltpu.VMEM_SHARED((NPAD, 16), f32),
    ],
    compiler_params=pltpu.CompilerParams(use_tc_tiling_on_sc=False),
)
def _deg_kernel(dst_hbm, ones_hbm, zrow_hbm, deg0_hbm, deg1_hbm,
                dst_v, ones_v, dacc_sh):
    cid = lax.axis_index("c")
    sid = lax.axis_index("s")
    base = sid * ROWS_PT
    pltpu.sync_copy(zrow_hbm, dacc_sh.at[pl.ds(base, ROWS_PT)])
    pltpu.sync_copy(dst_hbm.at[sid].at[pl.ds(cid * DEG_BLKS, DEG_BLKS)],
                    dst_v)
    pltpu.sync_copy(ones_hbm, ones_v)
    plsc.subcore_barrier()

    def body(j, c):
        pltpu.sync_copy(ones_v, dacc_sh.at[dst_v.at[j]], add=True)
        return c

    lax.fori_loop(0, DEG_BLKS, body, 0)
    plsc.subcore_barrier()

    @pl.when(cid == 0)
    def _():
        pltpu.sync_copy(dacc_sh.at[pl.ds(base, ROWS_PT)],
                        deg0_hbm.at[pl.ds(base, ROWS_PT)])

    @pl.when(cid == 1)
    def _():
        pltpu.sync_copy(dacc_sh.at[pl.ds(base, ROWS_PT)],
                        deg1_hbm.at[pl.ds(base, ROWS_PT)])


@functools.partial(
    pl.kernel,
    out_type=tuple(jax.ShapeDtypeStruct((NPAD, CW), f32) for _ in range(CH)),
    mesh=_mesh,
    scratch_types=[
        pltpu.VMEM((NBLK, BLK), jnp.int32),
        pltpu.VMEM((NBLK, BLK), jnp.int32),
        pltpu.VMEM((BLK, CW), f32),
        pltpu.VMEM((BLK, CW), f32),
        pltpu.VMEM((BLK, CW), f32),
        pltpu.VMEM((BLK, CW), f32),
        pltpu.VMEM((ZBLK, CW), f32),
        pltpu.VMEM_SHARED((NPAD, CW), f32),
        pltpu.SemaphoreType.DMA,
        pltpu.SemaphoreType.DMA,
    ],
    compiler_params=pltpu.CompilerParams(use_tc_tiling_on_sc=False),
)
def _agg_kernel(hp0, hp1, hp2, hp3, hp4, hp5, hp6, hp7,
                zrow_hbm, src_hbm, dst_hbm,
                a0, a1, a2, a3, a4, a5, a6, a7,
                src_v, dst_v, rows0_v, rows1_v, rows2_v, rows3_v,
                zero_v, acc_sh, sem_g, sem_s):
    cid = lax.axis_index("c")
    sid = lax.axis_index("s")
    base = sid * ROWS_PT
    pltpu.sync_copy(src_hbm.at[sid], src_v)
    pltpu.sync_copy(dst_hbm.at[sid], dst_v)
    pltpu.sync_copy(zrow_hbm, zero_v)
    rows = (rows0_v, rows1_v, rows2_v, rows3_v)

    def run_chunk(hp_hbm):
        for j in range(3):
            pltpu.async_copy(hp_hbm.at[src_v.at[j]], rows[j], sem_g)

        def group(p, c):
            for q in range(4):
                b = 4 * p + q
                buf = rows[q]
                pltpu.make_async_copy(hp_hbm.at[src_v.at[b]], buf,
                                      sem_g).wait()
                pltpu.async_copy(buf, acc_sh.at[dst_v.at[b]], sem_s,
                                 add=True)

                @pl.when(b >= 1)
                def _():
                    pltpu.make_async_copy(rows[(q + 3) % 4],
                                          acc_sh.at[dst_v.at[b]],
                                          sem_s).wait()

                @pl.when(b <= NBLK - 4)
                def _():
                    pltpu.async_copy(hp_hbm.at[src_v.at[b + 3]],
                                     rows[(q + 3) % 4], sem_g)
            return c

        lax.fori_loop(0, NBLK // 4, group, 0)
        pltpu.make_async_copy(rows[3], acc_sh.at[dst_v.at[NBLK - 1]],
                              sem_s).wait()

    hps = ((hp0, hp1, hp2, hp3), (hp4, hp5, hp6, hp7))
    outs = ((a0, a1, a2, a3), (a4, a5, a6, a7))
    for kc in range(CPS):
        for r in range(ZCHUNKS):
            pltpu.sync_copy(zero_v, acc_sh.at[pl.ds(base + r * ZBLK, ZBLK)])
        plsc.subcore_barrier()

        @pl.when(cid == 0)
        def _(kc=kc):
            run_chunk(hps[0][kc])

        @pl.when(cid == 1)
        def _(kc=kc):
            run_chunk(hps[1][kc])

        plsc.subcore_barrier()

        @pl.when(cid == 0)
        def _(kc=kc):
            pltpu.sync_copy(acc_sh.at[pl.ds(base, ROWS_PT)],
                            outs[0][kc].at[pl.ds(base, ROWS_PT)])

        @pl.when(cid == 1)
        def _(kc=kc):
            pltpu.sync_copy(acc_sh.at[pl.ds(base, ROWS_PT)],
                            outs[1][kc].at[pl.ds(base, ROWS_PT)])

        plsc.subcore_barrier()


NB = 1000
GRID = N // NB


def _k0_body(deg0, deg1, x, w0, dinv_out, *houts):
    deg = 1.0 + deg0[:, 0:1] + deg1[:, 0:1]
    dinv = lax.rsqrt(deg)
    dinv_out[...] = jnp.broadcast_to(dinv, (NB, 128))
    z = jnp.dot(x[...], w0[...], preferred_element_type=f32)
    hp = dinv * z
    for c, ref in enumerate(houts):
        ref[...] = hp[:, c * CW:(c + 1) * CW]


_k0 = pl.pallas_call(
    _k0_body,
    grid=(GRID,),
    in_specs=[
        pl.BlockSpec((NB, 16), lambda n: (n, 0)),
        pl.BlockSpec((NB, 16), lambda n: (n, 0)),
        pl.BlockSpec((NB, DIN), lambda n: (n, 0)),
        pl.BlockSpec((DIN, DH), lambda n: (0, 0)),
    ],
    out_specs=[pl.BlockSpec((NB, 128), lambda n: (n, 0))]
    + [pl.BlockSpec((NB, CW), lambda n: (n, 0))] * CH,
    out_shape=[jax.ShapeDtypeStruct((N, 128), f32)]
    + [jax.ShapeDtypeStruct((N, CW), f32)] * CH,
)


def _relu_rows(accs, hps, dv, b):
    parts = []
    for c in range(CH):
        u = (dv[:, :CW] * (accs[c][...] + hps[c][...])
             + b[:, c * CW:(c + 1) * CW])
        parts.append(jnp.maximum(u, 0.0))
    return jnp.concatenate(parts, axis=1)


def _mid_body(*refs):
    accs, hps = refs[:CH], refs[CH:2 * CH]
    dinv, b, w = refs[2 * CH], refs[2 * CH + 1], refs[2 * CH + 2]
    outs = refs[2 * CH + 3:]
    dv = dinv[...]
    u = _relu_rows(accs, hps, dv, b[...])
    z = jnp.dot(u, w[...], preferred_element_type=f32)
    for c, o in enumerate(outs):
        o[...] = dv[:, :CW] * z[:, c * CW:(c + 1) * CW]


_mid = pl.pallas_call(
    _mid_body,
    grid=(GRID,),
    in_specs=[pl.BlockSpec((NB, CW), lambda n: (n, 0))] * (2 * CH)
    + [pl.BlockSpec((NB, 128), lambda n: (n, 0)),
       pl.BlockSpec((1, DH), lambda n: (0, 0)),
       pl.BlockSpec((DH, DH), lambda n: (0, 0))],
    out_specs=[pl.BlockSpec((NB, CW), lambda n: (n, 0))] * CH,
    out_shape=[jax.ShapeDtypeStruct((N, CW), f32)] * CH,
)


def _fin_body(*refs):
    accs, hps = refs[:CH], refs[CH:2 * CH]
    dinv, b7 = refs[2 * CH], refs[2 * CH + 1]
    w0, b0, w1, b1, w2, b2 = refs[2 * CH + 2:2 * CH + 8]
    out = refs[2 * CH + 8]
    dv = dinv[...]
    u = _relu_rows(accs, hps, dv, b7[...])
    t = jnp.maximum(jnp.dot(u, w0[...], preferred_element_type=f32) + b0[...],
                    0.0)
    t = jnp.maximum(jnp.dot(t, w1[...], preferred_element_type=f32) + b1[...],
                    0.0)
    out[...] = jnp.dot(t, w2[...], preferred_element_type=f32) + b2[...]


_fin = pl.pallas_call(
    _fin_body,
    grid=(GRID,),
    in_specs=[pl.BlockSpec((NB, CW), lambda n: (n, 0))] * (2 * CH)
    + [pl.BlockSpec((NB, 128), lambda n: (n, 0)),
       pl.BlockSpec((1, DH), lambda n: (0, 0)),
       pl.BlockSpec((DH, DH), lambda n: (0, 0)),
       pl.BlockSpec((1, DH), lambda n: (0, 0)),
       pl.BlockSpec((DH, DH), lambda n: (0, 0)),
       pl.BlockSpec((1, DH), lambda n: (0, 0)),
       pl.BlockSpec((DH, NCLS), lambda n: (0, 0)),
       pl.BlockSpec((1, NCLS), lambda n: (0, 0))],
    out_specs=pl.BlockSpec((NB, NCLS), lambda n: (n, 0)),
    out_shape=jax.ShapeDtypeStruct((N, NCLS), f32),
)


def kernel(X, edge_index,
           conv_W0, conv_b0, conv_W1, conv_b1, conv_W2, conv_b2,
           conv_W3, conv_b3, conv_W4, conv_b4, conv_W5, conv_b5,
           conv_W6, conv_b6, conv_W7, conv_b7,
           cls_W0, cls_b0, cls_W1, cls_b1, cls_W2, cls_b2):
    Ws = [conv_W0, conv_W1, conv_W2, conv_W3,
          conv_W4, conv_W5, conv_W6, conv_W7]
    bs = [conv_b0, conv_b1, conv_b2, conv_b3,
          conv_b4, conv_b5, conv_b6, conv_b7]

    src_r = edge_index[0].reshape(NTILES, NBLK, BLK)
    dst_r = edge_index[1].reshape(NTILES, NBLK, BLK)
    ones_deg = jnp.ones((BLK, 16), f32)
    zdeg = jnp.zeros((ROWS_PT, 16), f32)
    zrow = jnp.zeros((ZBLK, CW), f32)

    deg0, deg1 = _deg_kernel(dst_r, ones_deg, zdeg)
    outs0 = _k0(deg0, deg1, X, Ws[0])
    dinv_b, hp = outs0[0], tuple(outs0[1:])
    for i in range(1, 8):
        acc = _agg_kernel(*hp, zrow, src_r, dst_r)
        hp = tuple(_mid(*acc, *hp, dinv_b, bs[i - 1].reshape(1, DH), Ws[i]))
    acc = _agg_kernel(*hp, zrow, src_r, dst_r)
    return _fin(*acc, *hp, dinv_b, bs[7].reshape(1, DH),
                cls_W0, cls_b0.reshape(1, DH),
                cls_W1, cls_b1.reshape(1, DH),
                cls_W2, cls_b2.reshape(1, NCLS))
